# Initial kernel scaffold; baseline (speedup 1.0000x reference)
import jax
import jax.numpy as jnp
from jax.experimental import pallas as pl


def kernel(x, edge_index, batchsize, edge_weight, gamma, beta, W4, b4, W5, b5, W6, b6, W1, b1, W2, b2):
    raise NotImplementedError("write your pallas kernel here")


# speedup vs baseline: 4.4613x; 4.4613x over previous
import functools

import jax
import jax.numpy as jnp
from jax import lax
from jax.experimental import pallas as pl
from jax.experimental.pallas import tpu as pltpu
from jax.experimental.pallas import tpu_sc as plsc

N = 10000
E = 160000
H = 256
F = 128
B = 64

NC = 2
NS = 16
L = 16

ET = E + N
ETP = 172032
ECHUNK = 128
NCHUNK = ETP // NS // ECHUNK
EBASE = ETP // NS
WCHUNK = ETP // (NC * NS) // ECHUNK

NP = 10112
ACC = 10240
DUMP = NP
RT = NP // NS
ZR = ACC // NS
PB = 72

_MESH = plsc.VectorSubcoreMesh(
    core_axis_name="c", subcore_axis_name="s", num_cores=NC, num_subcores=NS)


def _deg_body(dst_hbm, w_hbm, deg_hbm, deg_sh, zbuf, idxv, wv, gsem):
  cid = lax.axis_index("c")
  sid = lax.axis_index("s")
  wid = sid * NC + cid

  zv = jnp.zeros((L,), jnp.float32)
  for i in range(ZR // L):
    zbuf[pl.ds(i * L, L)] = zv
  pltpu.sync_copy(zbuf, deg_sh.at[pl.ds(pl.multiple_of(sid * ZR, 8), ZR)])
  plsc.subcore_barrier()

  def deg_chunk(g, carry):
    off = pl.multiple_of(wid * (ETP // (NC * NS)) + g * ECHUNK, ECHUNK)
    pltpu.sync_copy(dst_hbm.at[pl.ds(off, ECHUNK)], idxv)
    pltpu.sync_copy(w_hbm.at[pl.ds(off, ECHUNK)], wv)
    pltpu.sync_copy(wv, deg_sh.at[idxv], add=True)
    return carry
  lax.fori_loop(0, WCHUNK, deg_chunk, 0)
  plsc.subcore_barrier()

  zr0 = pl.multiple_of(sid * ZR, 8)
  pltpu.sync_copy(deg_sh.at[pl.ds(zr0, ZR)],
                  deg_hbm.at[cid, pl.ds(zr0, ZR)])


_deg = pl.kernel(
    _deg_body,
    out_type=jax.ShapeDtypeStruct((NC, ACC), jnp.float32),
    mesh=_MESH,
    scratch_types=[
        pltpu.VMEM_SHARED((ACC,), jnp.float32),
        pltpu.VMEM((ZR,), jnp.float32),
        pltpu.VMEM((ECHUNK,), jnp.int32),
        pltpu.VMEM((ECHUNK,), jnp.float32),
        pltpu.SemaphoreType.DMA,
    ],
)


def _make_prop(Hc):
  FV = Hc // L

  def body(xs_hbm, src_hbm, dst_hbm, w_hbm, bias_hbm, batch_hbm, dis_hbm,
           h_hbm, psum_hbm, pmax_hbm,
           acc_sh, batch_sh, dis_sh,
           zbuf, srcv, dstv, wv, rowsv, ebuf, biasv, psumv, pmaxv,
           tb4, rsum, rmax, bsm, dsm, gsem):
    cid = lax.axis_index("c")
    sid = lax.axis_index("s")

    @pl.when(sid == 0)
    def _():
      pltpu.sync_copy(batch_hbm, batch_sh)
      pltpu.sync_copy(dis_hbm, dis_sh)

    zv = jnp.zeros((L,), jnp.float32)
    for r in range(8):
      for f in range(FV):
        zbuf[r, pl.ds(f * L, L)] = zv

    def zero_chunk(z, carry):
      pltpu.sync_copy(zbuf, acc_sh.at[pl.ds(pl.multiple_of(sid * ZR + z * 8, 8), 8)])
      return carry
    lax.fori_loop(0, ZR // 8, zero_chunk, 0)

    ninf = jnp.full((L,), -jnp.inf, jnp.float32)
    def pinit(r, carry):
      for f in range(FV):
        sl = pl.ds(f * L, L)
        psumv[r, sl] = zv
        pmaxv[r, sl] = ninf
      return carry
    lax.fori_loop(0, PB, pinit, 0)

    pltpu.sync_copy(bias_hbm.at[cid], biasv)
    plsc.subcore_barrier()

    shift = cid * N

    def chunk(g, carry):
      off = pl.multiple_of(sid * EBASE + g * ECHUNK, ECHUNK)
      offw = pl.multiple_of(sid * (EBASE // L) + g * (ECHUNK // L), 8)
      pltpu.sync_copy(src_hbm.at[pl.ds(off, ECHUNK)], srcv)
      pltpu.sync_copy(dst_hbm.at[pl.ds(off, ECHUNK)], dstv)
      pltpu.sync_copy(w_hbm.at[pl.ds(offw, ECHUNK // L)], wv)
      for k in range(ECHUNK // L):
        sl = pl.ds(k * L, L)
        srcv[sl] = srcv[sl] + shift
      pltpu.async_copy(xs_hbm.at[srcv], rowsv, gsem).wait()

      def scale(m, c2):
        nv = wv[m]
        for j in range(L):
          s = lax.index_in_dim(nv, j, 0, keepdims=False)
          e = m * L + j
          for f in range(FV):
            sl = pl.ds(f * L, L)
            rowsv[e, sl] = rowsv[e, sl] * s
        return c2
      lax.fori_loop(0, ECHUNK // L, scale, 0)

      pltpu.sync_copy(rowsv, acc_sh.at[dstv], add=True)
      return carry
    lax.fori_loop(0, NCHUNK, chunk, 0)
    plsc.subcore_barrier()

    r0 = pl.multiple_of(sid * RT, 8)
    pltpu.sync_copy(batch_sh.at[pl.ds(r0, RT)], bsm)
    pltpu.sync_copy(dis_sh.at[pl.ds(r0, RT)], dsm)

    def echunk(z, carry):
      rz = pl.multiple_of(r0 + z * 8, 8)
      pltpu.sync_copy(acc_sh.at[pl.ds(rz, 8)], ebuf)
      for rr in range(8):
        r = z * 8 + rr
        b = bsm[r]
        d = dsm[r]
        for f in range(FV):
          sl = pl.ds(f * L, L)
          hv = jnp.maximum(ebuf[rr, sl] * d + biasv[sl], 0.0)
          ebuf[rr, sl] = hv
          psumv[b, sl] = psumv[b, sl] + hv
          pmaxv[b, sl] = jnp.maximum(pmaxv[b, sl], hv)
      pltpu.sync_copy(ebuf, h_hbm.at[cid, pl.ds(rz, 8)])
      return carry
    lax.fori_loop(0, RT // 8, echunk, 0)
    plsc.subcore_barrier()

    ps0 = pl.multiple_of(sid * B, 8)
    pltpu.sync_copy(psumv.at[pl.ds(0, B)], acc_sh.at[pl.ds(ps0, B)])
    pltpu.sync_copy(pmaxv.at[pl.ds(0, B)], acc_sh.at[pl.ds(1024 + ps0, B)])
    plsc.subcore_barrier()

    @pl.when(sid < 8)
    def _():
      s0 = pl.multiple_of(sid * (B // 8), 8)
      for q in range(B // 8):
        for fr in range(FV):
          sl = pl.ds(fr * L, L)
          rsum[q, sl] = zv
          rmax[q, sl] = ninf

      def red(p, carry):
        pltpu.sync_copy(acc_sh.at[pl.ds(pl.multiple_of(p * B + s0, 8), B // 8)], tb4)
        for q in range(B // 8):
          for fr in range(FV):
            sl = pl.ds(fr * L, L)
            rsum[q, sl] = rsum[q, sl] + tb4[q, sl]
        pltpu.sync_copy(acc_sh.at[pl.ds(pl.multiple_of(1024 + p * B + s0, 8), B // 8)], tb4)
        for q in range(B // 8):
          for fr in range(FV):
            sl = pl.ds(fr * L, L)
            rmax[q, sl] = jnp.maximum(rmax[q, sl], tb4[q, sl])
        return carry
      lax.fori_loop(0, NS, red, 0)

      pltpu.sync_copy(rsum, psum_hbm.at[cid, pl.ds(s0, B // 8)])
      pltpu.sync_copy(rmax, pmax_hbm.at[cid, pl.ds(s0, B // 8)])

  return pl.kernel(
      body,
      out_type=(
          jax.ShapeDtypeStruct((NC, NP, Hc), jnp.float32),
          jax.ShapeDtypeStruct((NC, B, Hc), jnp.float32),
          jax.ShapeDtypeStruct((NC, B, Hc), jnp.float32),
      ),
      mesh=_MESH,
      scratch_types=[
          pltpu.VMEM_SHARED((ACC, Hc), jnp.float32),
          pltpu.VMEM_SHARED((NP,), jnp.int32),
          pltpu.VMEM_SHARED((NP,), jnp.float32),
          pltpu.VMEM((8, Hc), jnp.float32),
          pltpu.VMEM((ECHUNK,), jnp.int32),
          pltpu.VMEM((ECHUNK,), jnp.int32),
          pltpu.VMEM((ECHUNK // L, L), jnp.float32),
          pltpu.VMEM((ECHUNK, Hc), jnp.float32),
          pltpu.VMEM((8, Hc), jnp.float32),
          pltpu.VMEM((Hc,), jnp.float32),
          pltpu.VMEM((PB, Hc), jnp.float32),
          pltpu.VMEM((PB, Hc), jnp.float32),
          pltpu.VMEM((B // 8, Hc), jnp.float32),
          pltpu.VMEM((B // 8, Hc), jnp.float32),
          pltpu.VMEM((B // 8, Hc), jnp.float32),
          pltpu.SMEM((RT,), jnp.int32),
          pltpu.SMEM((RT,), jnp.float32),
          pltpu.SemaphoreType.DMA,
      ],
  )


_prop128 = _make_prop(H // NC)


def _stats_body(x_ref, o_ref):
  xv = x_ref[...]
  s = jnp.sum(xv, axis=0)
  q = jnp.sum(xv * xv, axis=0)
  o_ref[...] = jnp.concatenate(
      [s[None], q[None], jnp.zeros((6, H), jnp.float32)], axis=0)


def _stats(x):
  return pl.pallas_call(
      _stats_body,
      out_shape=jax.ShapeDtypeStruct((8, H), jnp.float32),
  )(x)


def _dis_body(d_ref, o_ref):
  deg = d_ref[0] + d_ref[1]
  o_ref[...] = lax.rsqrt(jnp.maximum(deg, 1e-12))


def _dis(deg2):
  return pl.pallas_call(
      _dis_body,
      out_shape=jax.ShapeDtypeStruct((ACC // 128, 128), jnp.float32),
  )(deg2)


def _mm_body(n, x_ref, w_ref, st_ref, g_ref, bt_ref, dis_ref, o_ref):
  mu = st_ref[0:1] * (1.0 / n)
  msq = st_ref[1:2] * (1.0 / n)
  var = msq - mu * mu
  a = g_ref[...] * lax.rsqrt(var + 1e-5)
  c = bt_ref[...] - mu * a
  t = (x_ref[...] * a + c) * dis_ref[...]
  r = jnp.dot(t, w_ref[...], preferred_element_type=jnp.float32)
  hc = r.shape[1] // 2
  o_ref[0] = r[:, :hc]
  o_ref[1] = r[:, hc:]


def _mm(x, w, stats, gamma, beta, dis):
  n, k = x.shape
  ho = w.shape[1]
  hc = ho // 2
  rblk = 1000
  grid = n // rblk
  return pl.pallas_call(
      functools.partial(_mm_body, float(n)),
      grid=(grid,),
      in_specs=[
          pl.BlockSpec((rblk, k), lambda i: (i, 0)),
          pl.BlockSpec((k, ho), lambda i: (0, 0)),
          pl.BlockSpec((8, k), lambda i: (0, 0)),
          pl.BlockSpec((1, k), lambda i: (0, 0)),
          pl.BlockSpec((1, k), lambda i: (0, 0)),
          pl.BlockSpec((rblk, 1), lambda i: (i, 0)),
      ],
      out_specs=pl.BlockSpec((2, rblk, hc), lambda i: (0, i, 0)),
      out_shape=jax.ShapeDtypeStruct((2, n, hc), jnp.float32),
  )(x, w, stats, gamma, beta, dis)


def _final_body(s1, m1, s2, m2, s3, m3, bt, w1, b1, w2, b2, o_ref):
  seg = lax.broadcasted_iota(jnp.int32, (B, bt.shape[1]), 0)
  eq = (bt[...] == seg).astype(jnp.float32)
  cnt = jnp.sum(eq, axis=1, keepdims=True)
  rc = 1.0 / jnp.maximum(cnt, 1.0)
  xs = jnp.concatenate(
      [s1[...] * rc, m1[...], s2[...] * rc, m2[...], s3[...] * rc, m3[...]],
      axis=1)
  y = jnp.dot(xs, w1[...], preferred_element_type=jnp.float32) + b1[...]
  o_ref[...] = jnp.dot(y, w2[...], preferred_element_type=jnp.float32) + b2[...]


def _final(s1, m1, s2, m2, s3, m3, bt, w1, b1, w2p, b2p):
  return pl.pallas_call(
      _final_body,
      out_shape=jax.ShapeDtypeStruct((B, 128), jnp.float32),
  )(s1, m1, s2, m2, s3, m3, bt, w1, b1, w2p, b2p)


def kernel(x, edge_index, batchsize, edge_weight, gamma, beta,
           W4, b4, W5, b5, W6, b6, W1, b1, W2, b2):
  i32 = edge_index.dtype
  loop = jnp.arange(N, dtype=i32)
  src = jnp.concatenate(
      [edge_index[0], loop, jnp.zeros((ETP - ET,), i32)])
  dst = jnp.concatenate(
      [edge_index[1], loop, jnp.full((ETP - ET,), DUMP, i32)])
  w = jnp.concatenate(
      [edge_weight, jnp.ones((N,), jnp.float32),
       jnp.zeros((ETP - ET,), jnp.float32)])
  w2d = w.reshape(ETP // L, L)
  batch_p = jnp.concatenate(
      [batchsize.astype(jnp.int32), jnp.full((NP - N,), B, jnp.int32)])
  bt = jnp.concatenate(
      [batchsize.astype(jnp.int32), jnp.full((10240 - N,), B, jnp.int32)]
  ).reshape(1, 10240)

  ones_r = jnp.ones((1, H), jnp.float32)
  zeros_r = jnp.zeros((1, H), jnp.float32)
  stats_id = jnp.concatenate(
      [jnp.zeros((1, H), jnp.float32),
       jnp.full((1, H), float(N) * (1.0 - 1e-5), jnp.float32),
       jnp.zeros((6, H), jnp.float32)], axis=0)

  stats = _stats(x)
  deg2 = _deg(dst, w)
  dis = _dis(deg2.reshape(NC, ACC // 128, 128)).reshape(ACC)
  dis_col = dis[:N].reshape(N, 1)
  dis_np = dis[:NP]

  xw4 = _mm(x, W4, stats, gamma.reshape(1, H), beta.reshape(1, H), dis_col)
  h1, s1, m1 = _prop128(xw4.reshape(2 * N, H // 2), src, dst, w2d,
                        b4.reshape(2, H // 2), batch_p, dis_np)
  h1c = jnp.concatenate([h1[0, :N], h1[1, :N]], axis=1)

  xw5 = _mm(h1c, W5, stats_id, ones_r, zeros_r, dis_col)
  h2, s2, m2 = _prop128(xw5.reshape(2 * N, H // 2), src, dst, w2d,
                        b5.reshape(2, H // 2), batch_p, dis_np)
  h2c = jnp.concatenate([h2[0, :N], h2[1, :N]], axis=1)

  w6p = jnp.zeros((H, 2 * H // 2), jnp.float32)
  w6p = w6p.at[:, 0:F // 2].set(W6[:, :F // 2])
  w6p = w6p.at[:, H // 2:H // 2 + F // 2].set(W6[:, F // 2:])
  b6p = jnp.zeros((2, H // 2), jnp.float32)
  b6p = b6p.at[:, :F // 2].set(b6.reshape(2, F // 2))
  xw6 = _mm(h2c, w6p, stats_id, ones_r, zeros_r, dis_col)
  h3, s3, m3 = _prop128(xw6.reshape(2 * N, H // 2), src, dst, w2d,
                        b6p, batch_p, dis_np)
  h = jnp.concatenate([h3[0, :N, :F // 2], h3[1, :N, :F // 2]], axis=1)

  s1c = jnp.concatenate([s1[0], s1[1]], axis=1)
  m1c = jnp.concatenate([m1[0], m1[1]], axis=1)
  s2c = jnp.concatenate([s2[0], s2[1]], axis=1)
  m2c = jnp.concatenate([m2[0], m2[1]], axis=1)
  s3c = jnp.concatenate([s3[0][:, :F // 2], s3[1][:, :F // 2]], axis=1)
  m3c = jnp.concatenate([m3[0][:, :F // 2], m3[1][:, :F // 2]], axis=1)

  w2p = jnp.pad(W2, ((0, 0), (0, 127)))
  b2p = jnp.pad(b2.reshape(1, 1), ((0, 0), (0, 127)))
  yfull = _final(s1c, m1c, s2c, m2c, s3c, m3c, bt,
                 W1, b1.reshape(1, H), w2p, b2p)
  y_hat = yfull[:, :1]
  return (h, y_hat)


# speedup vs baseline: 6.9687x; 1.5620x over previous
import functools

import jax
import jax.numpy as jnp
from jax import lax
from jax.experimental import pallas as pl
from jax.experimental.pallas import tpu as pltpu
from jax.experimental.pallas import tpu_sc as plsc

N = 10000
E = 160000
H = 256
F = 128
B = 64

NC = 2
NS = 16
L = 16

ET = E + N
ETP = 172032
ECHUNK = 128
NCHUNK = ETP // NS // ECHUNK
EBASE = ETP // NS
WCHUNK = ETP // (NC * NS) // ECHUNK

NP = 10112
ACC = 10240
DUMP = NP
RT = NP // NS
ZR = ACC // NS
PB = 72

_MESH = plsc.VectorSubcoreMesh(
    core_axis_name="c", subcore_axis_name="s", num_cores=NC, num_subcores=NS)


def _deg_body(dst_hbm, w_hbm, deg_hbm, deg_sh, zbuf, idxv, wv, gsem):
  cid = lax.axis_index("c")
  sid = lax.axis_index("s")
  wid = sid * NC + cid

  zv = jnp.zeros((L,), jnp.float32)
  for i in range(ZR // L):
    zbuf[pl.ds(i * L, L)] = zv
  pltpu.sync_copy(zbuf, deg_sh.at[pl.ds(pl.multiple_of(sid * ZR, 8), ZR)])
  plsc.subcore_barrier()

  def deg_chunk(g, carry):
    off = pl.multiple_of(wid * (ETP // (NC * NS)) + g * ECHUNK, ECHUNK)
    pltpu.sync_copy(dst_hbm.at[pl.ds(off, ECHUNK)], idxv)
    pltpu.sync_copy(w_hbm.at[pl.ds(off, ECHUNK)], wv)
    pltpu.sync_copy(wv, deg_sh.at[idxv], add=True)
    return carry
  lax.fori_loop(0, WCHUNK, deg_chunk, 0)
  plsc.subcore_barrier()

  zr0 = pl.multiple_of(sid * ZR, 8)
  pltpu.sync_copy(deg_sh.at[pl.ds(zr0, ZR)],
                  deg_hbm.at[cid, pl.ds(zr0, ZR)])


_deg = pl.kernel(
    _deg_body,
    out_type=jax.ShapeDtypeStruct((NC, ACC), jnp.float32),
    mesh=_MESH,
    scratch_types=[
        pltpu.VMEM_SHARED((ACC,), jnp.float32),
        pltpu.VMEM((ZR,), jnp.float32),
        pltpu.VMEM((ECHUNK,), jnp.int32),
        pltpu.VMEM((ECHUNK,), jnp.float32),
        pltpu.SemaphoreType.DMA,
    ],
)


def _make_prop(Hc, FVe):

  def body(xs_hbm, src2_hbm, dst_hbm, w_hbm, bias_hbm, batch_hbm, dis_hbm,
           h_hbm, psum_hbm, pmax_hbm,
           acc_sh, pool_sh, batch_sh, dis_sh,
           zbuf, srcv0, srcv1, dstv0, dstv1, wv0, wv1, rowsv0, rowsv1,
           ebuf, bidx, biasv, pmaxv,
           bsm, dsm, isem0, isem1, gsem0, gsem1, ssem0, ssem1):
    cid = lax.axis_index("c")
    sid = lax.axis_index("s")
    FV = Hc // L

    @pl.when(sid == 0)
    def _():
      pltpu.sync_copy(batch_hbm, batch_sh)
      pltpu.sync_copy(dis_hbm, dis_sh)

    zv = jnp.zeros((L,), jnp.float32)
    for r in range(8):
      for f in range(FV):
        zbuf[r, pl.ds(f * L, L)] = zv

    def zero_chunk(z, carry):
      pltpu.sync_copy(zbuf, acc_sh.at[pl.ds(pl.multiple_of(sid * ZR + z * 8, 8), 8)])
      return carry
    lax.fori_loop(0, ZR // 8, zero_chunk, 0)

    @pl.when(sid == 0)
    def _():
      def zero_pool(z, carry):
        pltpu.sync_copy(zbuf, pool_sh.at[pl.ds(pl.multiple_of(z * 8, 8), 8)])
        return carry
      lax.fori_loop(0, PB // 8, zero_pool, 0)

    ninf = jnp.full((L,), -jnp.inf, jnp.float32)
    def pinit(r, carry):
      for f in range(FV):
        pmaxv[r, pl.ds(f * L, L)] = ninf
      return carry
    lax.fori_loop(0, PB, pinit, 0)

    pltpu.sync_copy(bias_hbm.at[cid], biasv)
    plsc.subcore_barrier()

    def do_scale(rowsref, wref):
      def scale(m, c2):
        nv = wref[m]
        for j in range(L):
          sj = lax.index_in_dim(nv, j, 0, keepdims=False)
          e = m * L + j
          for f in range(FVe):
            sl = pl.ds(f * L, L)
            rowsref[e, sl] = rowsref[e, sl] * sj
        return c2
      lax.fori_loop(0, ECHUNK // L, scale, 0)

    off0 = pl.multiple_of(sid * EBASE, ECHUNK)
    offw0 = pl.multiple_of(sid * (EBASE // L), 8)
    pltpu.sync_copy(src2_hbm.at[cid, pl.ds(off0, ECHUNK)], srcv0)
    pltpu.sync_copy(dst_hbm.at[pl.ds(off0, ECHUNK)], dstv0)
    pltpu.sync_copy(w_hbm.at[pl.ds(offw0, ECHUNK // L)], wv0)
    pltpu.async_copy(xs_hbm.at[srcv0], rowsv0, gsem0)

    bufs = ((srcv0, dstv0, wv0, rowsv0, gsem0, ssem0),
            (srcv1, dstv1, wv1, rowsv1, gsem1, ssem1))
    isems = (isem0, isem1)

    def pipe(g2, carry):
      for sslot in (0, 1):
        g = g2 * 2 + sslot
        srcv, dstv, wvs, rowsv, gsem, ssem = bufs[sslot]
        srcn, dstn, wvn, rowsn, gsemn, ssemn = bufs[1 - sslot]
        isn = isems[1 - sslot]
        offn = pl.multiple_of(sid * EBASE + (g + 1) * ECHUNK, ECHUNK)
        offwn = pl.multiple_of(sid * (EBASE // L) + (g + 1) * (ECHUNK // L), 8)

        @pl.when(g > 0)
        def _():
          pltpu.make_async_copy(rowsn, acc_sh.at[dstn], ssemn).wait()

        @pl.when(g + 1 < NCHUNK)
        def _():
          a = pltpu.async_copy(src2_hbm.at[cid, pl.ds(offn, ECHUNK)], srcn, isn)
          b = pltpu.async_copy(dst_hbm.at[pl.ds(offn, ECHUNK)], dstn, isn)
          c = pltpu.async_copy(w_hbm.at[pl.ds(offwn, ECHUNK // L)], wvn, isn)
          a.wait()
          b.wait()
          c.wait()
          pltpu.async_copy(xs_hbm.at[srcn], rowsn, gsemn)

        pltpu.make_async_copy(xs_hbm.at[srcv], rowsv, gsem).wait()
        do_scale(rowsv, wvs)
        pltpu.async_copy(rowsv, acc_sh.at[dstv], ssem, add=True)
      return carry
    lax.fori_loop(0, NCHUNK // 2, pipe, 0)
    pltpu.make_async_copy(rowsv1, acc_sh.at[dstv1], ssem1).wait()
    plsc.subcore_barrier()

    r0 = pl.multiple_of(sid * RT, 8)
    pltpu.sync_copy(batch_sh.at[pl.ds(r0, RT)], bsm)
    pltpu.sync_copy(dis_sh.at[pl.ds(r0, RT)], dsm)

    def echunk(z, carry):
      rz = pl.multiple_of(r0 + z * 8, 8)
      pltpu.sync_copy(acc_sh.at[pl.ds(rz, 8)], ebuf)
      pltpu.sync_copy(batch_sh.at[pl.ds(rz, 8)], bidx)
      for rr in range(8):
        r = z * 8 + rr
        d = dsm[r]
        for f in range(FVe):
          sl = pl.ds(f * L, L)
          hv = jnp.maximum(ebuf[rr, sl] * d + biasv[sl], 0.0)
          ebuf[rr, sl] = hv
        b = bsm[r]
        for f in range(FVe):
          sl = pl.ds(f * L, L)
          pmaxv[b, sl] = jnp.maximum(pmaxv[b, sl], ebuf[rr, sl])
      pltpu.sync_copy(ebuf, h_hbm.at[cid, pl.ds(rz, 8)])
      pltpu.sync_copy(ebuf, pool_sh.at[bidx], add=True)
      return carry
    lax.fori_loop(0, RT // 8, echunk, 0)
    plsc.subcore_barrier()

    ps0 = pl.multiple_of(sid * B, 8)
    pltpu.sync_copy(pmaxv.at[pl.ds(0, B)], acc_sh.at[pl.ds(ps0, B)])
    plsc.subcore_barrier()

    @pl.when(sid < 8)
    def _():
      s0 = pl.multiple_of(sid * (B // 8), 8)
      for q in range(8):
        for fr in range(FV):
          pmaxv[q, pl.ds(fr * L, L)] = ninf

      def red(pp, carry):
        pltpu.sync_copy(acc_sh.at[pl.ds(pl.multiple_of(pp * B + s0, 8), 8)], ebuf)
        for q in range(8):
          for fr in range(FVe):
            sl = pl.ds(fr * L, L)
            pmaxv[q, sl] = jnp.maximum(pmaxv[q, sl], ebuf[q, sl])
        return carry
      lax.fori_loop(0, NS, red, 0)

      pltpu.sync_copy(pmaxv.at[pl.ds(0, 8)], pmax_hbm.at[cid, pl.ds(s0, 8)])
      pltpu.sync_copy(pool_sh.at[pl.ds(s0, 8)], psum_hbm.at[cid, pl.ds(s0, 8)])

  return pl.kernel(
      body,
      out_type=(
          jax.ShapeDtypeStruct((NC, NP, Hc), jnp.float32),
          jax.ShapeDtypeStruct((NC, B, Hc), jnp.float32),
          jax.ShapeDtypeStruct((NC, B, Hc), jnp.float32),
      ),
      mesh=_MESH,
      scratch_types=[
          pltpu.VMEM_SHARED((ACC, Hc), jnp.float32),
          pltpu.VMEM_SHARED((PB, Hc), jnp.float32),
          pltpu.VMEM_SHARED((NP,), jnp.int32),
          pltpu.VMEM_SHARED((NP,), jnp.float32),
          pltpu.VMEM((8, Hc), jnp.float32),
          pltpu.VMEM((ECHUNK,), jnp.int32),
          pltpu.VMEM((ECHUNK,), jnp.int32),
          pltpu.VMEM((ECHUNK,), jnp.int32),
          pltpu.VMEM((ECHUNK,), jnp.int32),
          pltpu.VMEM((ECHUNK // L, L), jnp.float32),
          pltpu.VMEM((ECHUNK // L, L), jnp.float32),
          pltpu.VMEM((ECHUNK, Hc), jnp.float32),
          pltpu.VMEM((ECHUNK, Hc), jnp.float32),
          pltpu.VMEM((8, Hc), jnp.float32),
          pltpu.VMEM((8,), jnp.int32),
          pltpu.VMEM((Hc,), jnp.float32),
          pltpu.VMEM((PB, Hc), jnp.float32),
          pltpu.SMEM((RT,), jnp.int32),
          pltpu.SMEM((RT,), jnp.float32),
          pltpu.SemaphoreType.DMA,
          pltpu.SemaphoreType.DMA,
          pltpu.SemaphoreType.DMA,
          pltpu.SemaphoreType.DMA,
          pltpu.SemaphoreType.DMA,
          pltpu.SemaphoreType.DMA,
      ],
  )


_prop128 = _make_prop(H // NC, 8)
_prop64w = _make_prop(H // NC, 4)


def _stats_body(x_ref, o_ref):
  xv = x_ref[...]
  s = jnp.sum(xv, axis=0)
  q = jnp.sum(xv * xv, axis=0)
  o_ref[...] = jnp.concatenate(
      [s[None], q[None], jnp.zeros((6, H), jnp.float32)], axis=0)


def _stats(x):
  return pl.pallas_call(
      _stats_body,
      out_shape=jax.ShapeDtypeStruct((8, H), jnp.float32),
  )(x)


def _dis_body(d_ref, o_ref):
  deg = d_ref[0] + d_ref[1]
  o_ref[...] = lax.rsqrt(jnp.maximum(deg, 1e-12))


def _dis(deg2):
  return pl.pallas_call(
      _dis_body,
      out_shape=jax.ShapeDtypeStruct((ACC // 128, 128), jnp.float32),
  )(deg2)


def _mm_body(n, x_ref, w_ref, st_ref, g_ref, bt_ref, dis_ref, o_ref):
  mu = st_ref[0:1] * (1.0 / n)
  msq = st_ref[1:2] * (1.0 / n)
  var = msq - mu * mu
  a = g_ref[...] * lax.rsqrt(var + 1e-5)
  c = bt_ref[...] - mu * a
  t = (x_ref[...] * a + c) * dis_ref[...]
  r = jnp.dot(t, w_ref[...], preferred_element_type=jnp.float32)
  hc = r.shape[1] // 2
  o_ref[0] = r[:, :hc]
  o_ref[1] = r[:, hc:]


def _mm(x, w, stats, gamma, beta, dis):
  n, k = x.shape
  ho = w.shape[1]
  hc = ho // 2
  rblk = 1000
  grid = n // rblk
  return pl.pallas_call(
      functools.partial(_mm_body, float(n)),
      grid=(grid,),
      in_specs=[
          pl.BlockSpec((rblk, k), lambda i: (i, 0)),
          pl.BlockSpec((k, ho), lambda i: (0, 0)),
          pl.BlockSpec((8, k), lambda i: (0, 0)),
          pl.BlockSpec((1, k), lambda i: (0, 0)),
          pl.BlockSpec((1, k), lambda i: (0, 0)),
          pl.BlockSpec((rblk, 1), lambda i: (i, 0)),
      ],
      out_specs=pl.BlockSpec((2, rblk, hc), lambda i: (0, i, 0)),
      out_shape=jax.ShapeDtypeStruct((2, n, hc), jnp.float32),
  )(x, w, stats, gamma, beta, dis)


def _final_body(s1, m1, s2, m2, s3, m3, bt, w1, b1, w2, b2, o_ref):
  seg = lax.broadcasted_iota(jnp.int32, (B, bt.shape[1]), 0)
  eq = (bt[...] == seg).astype(jnp.float32)
  cnt = jnp.sum(eq, axis=1, keepdims=True)
  rc = 1.0 / jnp.maximum(cnt, 1.0)
  xs = jnp.concatenate(
      [s1[...] * rc, m1[...], s2[...] * rc, m2[...], s3[...] * rc, m3[...]],
      axis=1)
  y = jnp.dot(xs, w1[...], preferred_element_type=jnp.float32) + b1[...]
  o_ref[...] = jnp.dot(y, w2[...], preferred_element_type=jnp.float32) + b2[...]


def _final(s1, m1, s2, m2, s3, m3, bt, w1, b1, w2p, b2p):
  return pl.pallas_call(
      _final_body,
      out_shape=jax.ShapeDtypeStruct((B, 128), jnp.float32),
  )(s1, m1, s2, m2, s3, m3, bt, w1, b1, w2p, b2p)


def kernel(x, edge_index, batchsize, edge_weight, gamma, beta,
           W4, b4, W5, b5, W6, b6, W1, b1, W2, b2):
  i32 = edge_index.dtype
  loop = jnp.arange(N, dtype=i32)
  src = jnp.concatenate(
      [edge_index[0], loop, jnp.zeros((ETP - ET,), i32)])
  src2 = jnp.stack([src, src + N])
  dst = jnp.concatenate(
      [edge_index[1], loop, jnp.full((ETP - ET,), DUMP, i32)])
  w = jnp.concatenate(
      [edge_weight, jnp.ones((N,), jnp.float32),
       jnp.zeros((ETP - ET,), jnp.float32)])
  w2d = w.reshape(ETP // L, L)
  batch_p = jnp.concatenate(
      [batchsize.astype(jnp.int32), jnp.full((NP - N,), B, jnp.int32)])
  bt = jnp.concatenate(
      [batchsize.astype(jnp.int32), jnp.full((10240 - N,), B, jnp.int32)]
  ).reshape(1, 10240)

  ones_r = jnp.ones((1, H), jnp.float32)
  zeros_r = jnp.zeros((1, H), jnp.float32)
  stats_id = jnp.concatenate(
      [jnp.zeros((1, H), jnp.float32),
       jnp.full((1, H), float(N) * (1.0 - 1e-5), jnp.float32),
       jnp.zeros((6, H), jnp.float32)], axis=0)

  stats = _stats(x)
  deg2 = _deg(dst, w)
  dis = _dis(deg2.reshape(NC, ACC // 128, 128)).reshape(ACC)
  dis_col = dis[:N].reshape(N, 1)
  dis_np = dis[:NP]

  xw4 = _mm(x, W4, stats, gamma.reshape(1, H), beta.reshape(1, H), dis_col)
  h1, s1, m1 = _prop128(xw4.reshape(2 * N, H // 2), src2, dst, w2d,
                        b4.reshape(2, H // 2), batch_p, dis_np)
  h1c = jnp.concatenate([h1[0, :N], h1[1, :N]], axis=1)

  xw5 = _mm(h1c, W5, stats_id, ones_r, zeros_r, dis_col)
  h2, s2, m2 = _prop128(xw5.reshape(2 * N, H // 2), src2, dst, w2d,
                        b5.reshape(2, H // 2), batch_p, dis_np)
  h2c = jnp.concatenate([h2[0, :N], h2[1, :N]], axis=1)

  w6p = jnp.zeros((H, 2 * H // 2), jnp.float32)
  w6p = w6p.at[:, 0:F // 2].set(W6[:, :F // 2])
  w6p = w6p.at[:, H // 2:H // 2 + F // 2].set(W6[:, F // 2:])
  b6p = jnp.zeros((2, H // 2), jnp.float32)
  b6p = b6p.at[:, :F // 2].set(b6.reshape(2, F // 2))
  xw6 = _mm(h2c, w6p, stats_id, ones_r, zeros_r, dis_col)
  h3, s3, m3 = _prop64w(xw6.reshape(2 * N, H // 2), src2, dst, w2d,
                        b6p, batch_p, dis_np)
  h = jnp.concatenate([h3[0, :N, :F // 2], h3[1, :N, :F // 2]], axis=1)

  s1c = jnp.concatenate([s1[0], s1[1]], axis=1)
  m1c = jnp.concatenate([m1[0], m1[1]], axis=1)
  s2c = jnp.concatenate([s2[0], s2[1]], axis=1)
  m2c = jnp.concatenate([m2[0], m2[1]], axis=1)
  s3c = jnp.concatenate([s3[0][:, :F // 2], s3[1][:, :F // 2]], axis=1)
  m3c = jnp.concatenate([m3[0][:, :F // 2], m3[1][:, :F // 2]], axis=1)

  w2p = jnp.pad(W2, ((0, 0), (0, 127)))
  b2p = jnp.pad(b2.reshape(1, 1), ((0, 0), (0, 127)))
  yfull = _final(s1c, m1c, s2c, m2c, s3c, m3c, bt,
                 W1, b1.reshape(1, H), w2p, b2p)
  y_hat = yfull[:, :1]
  return (h, y_hat)


# speedup vs baseline: 6.9811x; 1.0018x over previous
import functools

import jax
import jax.numpy as jnp
from jax import lax
from jax.experimental import pallas as pl
from jax.experimental.pallas import tpu as pltpu
from jax.experimental.pallas import tpu_sc as plsc

N = 10000
E = 160000
H = 256
F = 128
B = 64

NC = 2
NS = 16
L = 16

ET = E + N
ETP = 172032
ECHUNK = 128
NCHUNK = ETP // NS // ECHUNK
EBASE = ETP // NS
WCHUNK = ETP // (NC * NS) // ECHUNK

NP = 10112
ACC = 10240
DUMP = NP
RT = NP // NS
ZR = ACC // NS
PB = 72

_MESH = plsc.VectorSubcoreMesh(
    core_axis_name="c", subcore_axis_name="s", num_cores=NC, num_subcores=NS)


def _deg_body(dst_hbm, w_hbm, deg_hbm, deg_sh, zbuf, idxv, wv, gsem):
  cid = lax.axis_index("c")
  sid = lax.axis_index("s")
  wid = sid * NC + cid

  zv = jnp.zeros((L,), jnp.float32)
  for i in range(ZR // L):
    zbuf[pl.ds(i * L, L)] = zv
  pltpu.sync_copy(zbuf, deg_sh.at[pl.ds(pl.multiple_of(sid * ZR, 8), ZR)])
  plsc.subcore_barrier()

  def deg_chunk(g, carry):
    off = pl.multiple_of(wid * (ETP // (NC * NS)) + g * ECHUNK, ECHUNK)
    pltpu.sync_copy(dst_hbm.at[pl.ds(off, ECHUNK)], idxv)
    pltpu.sync_copy(w_hbm.at[pl.ds(off, ECHUNK)], wv)
    pltpu.sync_copy(wv, deg_sh.at[idxv], add=True)
    return carry
  lax.fori_loop(0, WCHUNK, deg_chunk, 0)
  plsc.subcore_barrier()

  zr0 = pl.multiple_of(sid * ZR, 8)
  pltpu.sync_copy(deg_sh.at[pl.ds(zr0, ZR)],
                  deg_hbm.at[cid, pl.ds(zr0, ZR)])


_deg = pl.kernel(
    _deg_body,
    out_type=jax.ShapeDtypeStruct((NC, ACC), jnp.float32),
    mesh=_MESH,
    scratch_types=[
        pltpu.VMEM_SHARED((ACC,), jnp.float32),
        pltpu.VMEM((ZR,), jnp.float32),
        pltpu.VMEM((ECHUNK,), jnp.int32),
        pltpu.VMEM((ECHUNK,), jnp.float32),
        pltpu.SemaphoreType.DMA,
    ],
)


def _make_prop(Hc, FVe):

  def body(xs_hbm, src2_hbm, dst_hbm, w_hbm, bias_hbm, batch_hbm, dis_hbm,
           h_hbm, psum_hbm, pmax_hbm,
           acc_sh, pool_sh, batch_sh, dis_sh,
           zbuf, srcv0, srcv1, dstv0, dstv1, wv0, wv1, rowsv0, rowsv1,
           ebuf, bidx, biasv, pmaxv,
           bsm, dsm, isem0, isem1, gsem0, gsem1, ssem0, ssem1):
    cid = lax.axis_index("c")
    sid = lax.axis_index("s")
    FV = Hc // L

    @pl.when(sid == 0)
    def _():
      pltpu.sync_copy(batch_hbm, batch_sh)
      pltpu.sync_copy(dis_hbm, dis_sh)

    zv = jnp.zeros((L,), jnp.float32)
    for r in range(8):
      for f in range(FV):
        zbuf[r, pl.ds(f * L, L)] = zv

    def zero_chunk(z, carry):
      pltpu.sync_copy(zbuf, acc_sh.at[pl.ds(pl.multiple_of(sid * ZR + z * 8, 8), 8)])
      return carry
    lax.fori_loop(0, ZR // 8, zero_chunk, 0)

    @pl.when(sid == 0)
    def _():
      def zero_pool(z, carry):
        pltpu.sync_copy(zbuf, pool_sh.at[pl.ds(pl.multiple_of(z * 8, 8), 8)])
        return carry
      lax.fori_loop(0, PB // 8, zero_pool, 0)

    ninf = jnp.full((L,), -jnp.inf, jnp.float32)
    def pinit(r, carry):
      for f in range(FV):
        pmaxv[r, pl.ds(f * L, L)] = ninf
      return carry
    lax.fori_loop(0, PB, pinit, 0)

    pltpu.sync_copy(bias_hbm.at[cid], biasv)
    plsc.subcore_barrier()

    def do_scale(rowsref, wref):
      @plsc.parallel_loop(0, ECHUNK // L, step=1, unroll=2)
      def _(m):
        nv = wref[m]
        for j in range(L):
          sj = lax.index_in_dim(nv, j, 0, keepdims=False)
          e = m * L + j
          for f in range(FVe):
            sl = pl.ds(f * L, L)
            rowsref[e, sl] = rowsref[e, sl] * sj

    off0 = pl.multiple_of(sid * EBASE, ECHUNK)
    offw0 = pl.multiple_of(sid * (EBASE // L), 8)
    pltpu.sync_copy(src2_hbm.at[cid, pl.ds(off0, ECHUNK)], srcv0)
    pltpu.sync_copy(dst_hbm.at[pl.ds(off0, ECHUNK)], dstv0)
    pltpu.sync_copy(w_hbm.at[pl.ds(offw0, ECHUNK // L)], wv0)
    pltpu.async_copy(xs_hbm.at[srcv0], rowsv0, gsem0)

    bufs = ((srcv0, dstv0, wv0, rowsv0, gsem0, ssem0),
            (srcv1, dstv1, wv1, rowsv1, gsem1, ssem1))
    isems = (isem0, isem1)

    def pipe(g2, carry):
      for sslot in (0, 1):
        g = g2 * 2 + sslot
        srcv, dstv, wvs, rowsv, gsem, ssem = bufs[sslot]
        srcn, dstn, wvn, rowsn, gsemn, ssemn = bufs[1 - sslot]
        isn = isems[1 - sslot]
        offn = pl.multiple_of(sid * EBASE + (g + 1) * ECHUNK, ECHUNK)
        offwn = pl.multiple_of(sid * (EBASE // L) + (g + 1) * (ECHUNK // L), 8)

        @pl.when(g > 0)
        def _():
          pltpu.make_async_copy(rowsn, acc_sh.at[dstn], ssemn).wait()

        @pl.when(g + 1 < NCHUNK)
        def _():
          a = pltpu.async_copy(src2_hbm.at[cid, pl.ds(offn, ECHUNK)], srcn, isn)
          b = pltpu.async_copy(dst_hbm.at[pl.ds(offn, ECHUNK)], dstn, isn)
          c = pltpu.async_copy(w_hbm.at[pl.ds(offwn, ECHUNK // L)], wvn, isn)
          a.wait()
          b.wait()
          c.wait()
          pltpu.async_copy(xs_hbm.at[srcn], rowsn, gsemn)

        pltpu.make_async_copy(xs_hbm.at[srcv], rowsv, gsem).wait()
        do_scale(rowsv, wvs)
        pltpu.async_copy(rowsv, acc_sh.at[dstv], ssem, add=True)
      return carry
    lax.fori_loop(0, NCHUNK // 2, pipe, 0)
    pltpu.make_async_copy(rowsv1, acc_sh.at[dstv1], ssem1).wait()
    plsc.subcore_barrier()

    r0 = pl.multiple_of(sid * RT, 8)
    pltpu.sync_copy(batch_sh.at[pl.ds(r0, RT)], bsm)
    pltpu.sync_copy(dis_sh.at[pl.ds(r0, RT)], dsm)

    def echunk(z, carry):
      rz = pl.multiple_of(r0 + z * 8, 8)
      pltpu.sync_copy(acc_sh.at[pl.ds(rz, 8)], ebuf)
      pltpu.sync_copy(batch_sh.at[pl.ds(rz, 8)], bidx)
      for rr in range(8):
        r = z * 8 + rr
        d = dsm[r]
        for f in range(FVe):
          sl = pl.ds(f * L, L)
          hv = jnp.maximum(ebuf[rr, sl] * d + biasv[sl], 0.0)
          ebuf[rr, sl] = hv
        b = bsm[r]
        for f in range(FVe):
          sl = pl.ds(f * L, L)
          pmaxv[b, sl] = jnp.maximum(pmaxv[b, sl], ebuf[rr, sl])
      pltpu.sync_copy(ebuf, h_hbm.at[cid, pl.ds(rz, 8)])
      pltpu.sync_copy(ebuf, pool_sh.at[bidx], add=True)
      return carry
    lax.fori_loop(0, RT // 8, echunk, 0)
    plsc.subcore_barrier()

    ps0 = pl.multiple_of(sid * B, 8)
    pltpu.sync_copy(pmaxv.at[pl.ds(0, B)], acc_sh.at[pl.ds(ps0, B)])
    plsc.subcore_barrier()

    @pl.when(sid < 8)
    def _():
      s0 = pl.multiple_of(sid * (B // 8), 8)
      for q in range(8):
        for fr in range(FV):
          pmaxv[q, pl.ds(fr * L, L)] = ninf

      def red(pp, carry):
        pltpu.sync_copy(acc_sh.at[pl.ds(pl.multiple_of(pp * B + s0, 8), 8)], ebuf)
        for q in range(8):
          for fr in range(FVe):
            sl = pl.ds(fr * L, L)
            pmaxv[q, sl] = jnp.maximum(pmaxv[q, sl], ebuf[q, sl])
        return carry
      lax.fori_loop(0, NS, red, 0)

      pltpu.sync_copy(pmaxv.at[pl.ds(0, 8)], pmax_hbm.at[cid, pl.ds(s0, 8)])
      pltpu.sync_copy(pool_sh.at[pl.ds(s0, 8)], psum_hbm.at[cid, pl.ds(s0, 8)])

  return pl.kernel(
      body,
      out_type=(
          jax.ShapeDtypeStruct((NC, NP, Hc), jnp.float32),
          jax.ShapeDtypeStruct((NC, B, Hc), jnp.float32),
          jax.ShapeDtypeStruct((NC, B, Hc), jnp.float32),
      ),
      mesh=_MESH,
      scratch_types=[
          pltpu.VMEM_SHARED((ACC, Hc), jnp.float32),
          pltpu.VMEM_SHARED((PB, Hc), jnp.float32),
          pltpu.VMEM_SHARED((NP,), jnp.int32),
          pltpu.VMEM_SHARED((NP,), jnp.float32),
          pltpu.VMEM((8, Hc), jnp.float32),
          pltpu.VMEM((ECHUNK,), jnp.int32),
          pltpu.VMEM((ECHUNK,), jnp.int32),
          pltpu.VMEM((ECHUNK,), jnp.int32),
          pltpu.VMEM((ECHUNK,), jnp.int32),
          pltpu.VMEM((ECHUNK // L, L), jnp.float32),
          pltpu.VMEM((ECHUNK // L, L), jnp.float32),
          pltpu.VMEM((ECHUNK, Hc), jnp.float32),
          pltpu.VMEM((ECHUNK, Hc), jnp.float32),
          pltpu.VMEM((8, Hc), jnp.float32),
          pltpu.VMEM((8,), jnp.int32),
          pltpu.VMEM((Hc,), jnp.float32),
          pltpu.VMEM((PB, Hc), jnp.float32),
          pltpu.SMEM((RT,), jnp.int32),
          pltpu.SMEM((RT,), jnp.float32),
          pltpu.SemaphoreType.DMA,
          pltpu.SemaphoreType.DMA,
          pltpu.SemaphoreType.DMA,
          pltpu.SemaphoreType.DMA,
          pltpu.SemaphoreType.DMA,
          pltpu.SemaphoreType.DMA,
      ],
  )


_prop128 = _make_prop(H // NC, 8)
_prop64w = _make_prop(H // NC, 4)


def _stats_body(x_ref, o_ref):
  xv = x_ref[...]
  s = jnp.sum(xv, axis=0)
  q = jnp.sum(xv * xv, axis=0)
  o_ref[...] = jnp.concatenate(
      [s[None], q[None], jnp.zeros((6, H), jnp.float32)], axis=0)


def _stats(x):
  return pl.pallas_call(
      _stats_body,
      out_shape=jax.ShapeDtypeStruct((8, H), jnp.float32),
  )(x)


def _dis_body(d_ref, o_ref):
  deg = d_ref[0] + d_ref[1]
  o_ref[...] = lax.rsqrt(jnp.maximum(deg, 1e-12))


def _dis(deg2):
  return pl.pallas_call(
      _dis_body,
      out_shape=jax.ShapeDtypeStruct((ACC // 128, 128), jnp.float32),
  )(deg2)


def _mm_body(n, x_ref, w_ref, st_ref, g_ref, bt_ref, dis_ref, o_ref):
  mu = st_ref[0:1] * (1.0 / n)
  msq = st_ref[1:2] * (1.0 / n)
  var = msq - mu * mu
  a = g_ref[...] * lax.rsqrt(var + 1e-5)
  c = bt_ref[...] - mu * a
  t = (x_ref[...] * a + c) * dis_ref[...]
  r = jnp.dot(t, w_ref[...], preferred_element_type=jnp.float32)
  hc = r.shape[1] // 2
  o_ref[0] = r[:, :hc]
  o_ref[1] = r[:, hc:]


def _mm(x, w, stats, gamma, beta, dis):
  n, k = x.shape
  ho = w.shape[1]
  hc = ho // 2
  rblk = 1000
  grid = n // rblk
  return pl.pallas_call(
      functools.partial(_mm_body, float(n)),
      grid=(grid,),
      in_specs=[
          pl.BlockSpec((rblk, k), lambda i: (i, 0)),
          pl.BlockSpec((k, ho), lambda i: (0, 0)),
          pl.BlockSpec((8, k), lambda i: (0, 0)),
          pl.BlockSpec((1, k), lambda i: (0, 0)),
          pl.BlockSpec((1, k), lambda i: (0, 0)),
          pl.BlockSpec((rblk, 1), lambda i: (i, 0)),
      ],
      out_specs=pl.BlockSpec((2, rblk, hc), lambda i: (0, i, 0)),
      out_shape=jax.ShapeDtypeStruct((2, n, hc), jnp.float32),
  )(x, w, stats, gamma, beta, dis)


def _final_body(s1, m1, s2, m2, s3, m3, bt, w1, b1, w2, b2, o_ref):
  seg = lax.broadcasted_iota(jnp.int32, (B, bt.shape[1]), 0)
  eq = (bt[...] == seg).astype(jnp.float32)
  cnt = jnp.sum(eq, axis=1, keepdims=True)
  rc = 1.0 / jnp.maximum(cnt, 1.0)
  xs = jnp.concatenate(
      [s1[...] * rc, m1[...], s2[...] * rc, m2[...], s3[...] * rc, m3[...]],
      axis=1)
  y = jnp.dot(xs, w1[...], preferred_element_type=jnp.float32) + b1[...]
  o_ref[...] = jnp.dot(y, w2[...], preferred_element_type=jnp.float32) + b2[...]


def _final(s1, m1, s2, m2, s3, m3, bt, w1, b1, w2p, b2p):
  return pl.pallas_call(
      _final_body,
      out_shape=jax.ShapeDtypeStruct((B, 128), jnp.float32),
  )(s1, m1, s2, m2, s3, m3, bt, w1, b1, w2p, b2p)


def kernel(x, edge_index, batchsize, edge_weight, gamma, beta,
           W4, b4, W5, b5, W6, b6, W1, b1, W2, b2):
  i32 = edge_index.dtype
  loop = jnp.arange(N, dtype=i32)
  src = jnp.concatenate(
      [edge_index[0], loop, jnp.zeros((ETP - ET,), i32)])
  src2 = jnp.stack([src, src + N])
  dst = jnp.concatenate(
      [edge_index[1], loop, jnp.full((ETP - ET,), DUMP, i32)])
  w = jnp.concatenate(
      [edge_weight, jnp.ones((N,), jnp.float32),
       jnp.zeros((ETP - ET,), jnp.float32)])
  w2d = w.reshape(ETP // L, L)
  batch_p = jnp.concatenate(
      [batchsize.astype(jnp.int32), jnp.full((NP - N,), B, jnp.int32)])
  bt = jnp.concatenate(
      [batchsize.astype(jnp.int32), jnp.full((10240 - N,), B, jnp.int32)]
  ).reshape(1, 10240)

  ones_r = jnp.ones((1, H), jnp.float32)
  zeros_r = jnp.zeros((1, H), jnp.float32)
  stats_id = jnp.concatenate(
      [jnp.zeros((1, H), jnp.float32),
       jnp.full((1, H), float(N) * (1.0 - 1e-5), jnp.float32),
       jnp.zeros((6, H), jnp.float32)], axis=0)

  stats = _stats(x)
  deg2 = _deg(dst, w)
  dis = _dis(deg2.reshape(NC, ACC // 128, 128)).reshape(ACC)
  dis_col = dis[:N].reshape(N, 1)
  dis_np = dis[:NP]

  xw4 = _mm(x, W4, stats, gamma.reshape(1, H), beta.reshape(1, H), dis_col)
  h1, s1, m1 = _prop128(xw4.reshape(2 * N, H // 2), src2, dst, w2d,
                        b4.reshape(2, H // 2), batch_p, dis_np)
  h1c = jnp.concatenate([h1[0, :N], h1[1, :N]], axis=1)

  xw5 = _mm(h1c, W5, stats_id, ones_r, zeros_r, dis_col)
  h2, s2, m2 = _prop128(xw5.reshape(2 * N, H // 2), src2, dst, w2d,
                        b5.reshape(2, H // 2), batch_p, dis_np)
  h2c = jnp.concatenate([h2[0, :N], h2[1, :N]], axis=1)

  w6p = jnp.zeros((H, 2 * H // 2), jnp.float32)
  w6p = w6p.at[:, 0:F // 2].set(W6[:, :F // 2])
  w6p = w6p.at[:, H // 2:H // 2 + F // 2].set(W6[:, F // 2:])
  b6p = jnp.zeros((2, H // 2), jnp.float32)
  b6p = b6p.at[:, :F // 2].set(b6.reshape(2, F // 2))
  xw6 = _mm(h2c, w6p, stats_id, ones_r, zeros_r, dis_col)
  h3, s3, m3 = _prop64w(xw6.reshape(2 * N, H // 2), src2, dst, w2d,
                        b6p, batch_p, dis_np)
  h = jnp.concatenate([h3[0, :N, :F // 2], h3[1, :N, :F // 2]], axis=1)

  s1c = jnp.concatenate([s1[0], s1[1]], axis=1)
  m1c = jnp.concatenate([m1[0], m1[1]], axis=1)
  s2c = jnp.concatenate([s2[0], s2[1]], axis=1)
  m2c = jnp.concatenate([m2[0], m2[1]], axis=1)
  s3c = jnp.concatenate([s3[0][:, :F // 2], s3[1][:, :F // 2]], axis=1)
  m3c = jnp.concatenate([m3[0][:, :F // 2], m3[1][:, :F // 2]], axis=1)

  w2p = jnp.pad(W2, ((0, 0), (0, 127)))
  b2p = jnp.pad(b2.reshape(1, 1), ((0, 0), (0, 127)))
  yfull = _final(s1c, m1c, s2c, m2c, s3c, m3c, bt,
                 W1, b1.reshape(1, H), w2p, b2p)
  y_hat = yfull[:, :1]
  return (h, y_hat)


# speedup vs baseline: 7.5260x; 1.0780x over previous
import functools

import jax
import jax.numpy as jnp
from jax import lax
from jax.experimental import pallas as pl
from jax.experimental.pallas import tpu as pltpu
from jax.experimental.pallas import tpu_sc as plsc

N = 10000
E = 160000
H = 256
F = 128
B = 64

NC = 2
NS = 16
L = 16

ET = E + N
ETP = 172032
ECHUNK = 128
NCHUNK = ETP // NS // ECHUNK
EBASE = ETP // NS
WCHUNK = ETP // (NC * NS) // ECHUNK

NP = 10112
ACC = 10240
DUMP = NP
RT = NP // NS
ZR = ACC // NS
PB = 72

_MESH = plsc.VectorSubcoreMesh(
    core_axis_name="c", subcore_axis_name="s", num_cores=NC, num_subcores=NS)


def _deg_body(dst_hbm, w_hbm, deg_hbm, deg_sh, zbuf, idxv, wv, gsem):
  cid = lax.axis_index("c")
  sid = lax.axis_index("s")
  wid = sid * NC + cid

  zv = jnp.zeros((L,), jnp.float32)
  for i in range(ZR // L):
    zbuf[pl.ds(i * L, L)] = zv
  pltpu.sync_copy(zbuf, deg_sh.at[pl.ds(pl.multiple_of(sid * ZR, 8), ZR)])
  plsc.subcore_barrier()

  def deg_chunk(g, carry):
    off = pl.multiple_of(wid * (ETP // (NC * NS)) + g * ECHUNK, ECHUNK)
    pltpu.sync_copy(dst_hbm.at[pl.ds(off, ECHUNK)], idxv)
    pltpu.sync_copy(w_hbm.at[pl.ds(off, ECHUNK)], wv)
    pltpu.sync_copy(wv, deg_sh.at[idxv], add=True)
    return carry
  lax.fori_loop(0, WCHUNK, deg_chunk, 0)
  plsc.subcore_barrier()

  zr0 = pl.multiple_of(sid * ZR, 8)
  pltpu.sync_copy(deg_sh.at[pl.ds(zr0, ZR)],
                  deg_hbm.at[cid, pl.ds(zr0, ZR)])


_deg = pl.kernel(
    _deg_body,
    out_type=jax.ShapeDtypeStruct((NC, ACC), jnp.float32),
    mesh=_MESH,
    scratch_types=[
        pltpu.VMEM_SHARED((ACC,), jnp.float32),
        pltpu.VMEM((ZR,), jnp.float32),
        pltpu.VMEM((ECHUNK,), jnp.int32),
        pltpu.VMEM((ECHUNK,), jnp.float32),
        pltpu.SemaphoreType.DMA,
    ],
)


def _make_prop(Hc, FVe):

  def body(xs_hbm, src2_hbm, dst_hbm, w_hbm, bias_hbm, batch_hbm, dis_hbm,
           h_hbm, psum_hbm, pmax_hbm,
           acc_sh, pool_sh, batch_sh, dis_sh,
           zbuf, srcv0, srcv1, dstv0, dstv1, wv0, wv1, rowsv0, rowsv1,
           ebuf, bidx, biasv, pmaxv,
           bsm, dsm, isem0, isem1, gsem0, gsem1, ssem0, ssem1):
    cid = lax.axis_index("c")
    sid = lax.axis_index("s")
    FV = Hc // L

    @pl.when(sid == 0)
    def _():
      pltpu.sync_copy(batch_hbm, batch_sh)
      pltpu.sync_copy(dis_hbm, dis_sh)

    zv = jnp.zeros((L,), jnp.float32)
    for r in range(8):
      for f in range(FV):
        zbuf[r, pl.ds(f * L, L)] = zv

    def zero_chunk(z, carry):
      pltpu.sync_copy(zbuf, acc_sh.at[pl.ds(pl.multiple_of(sid * ZR + z * 8, 8), 8)])
      return carry
    lax.fori_loop(0, ZR // 8, zero_chunk, 0)

    @pl.when(sid == 0)
    def _():
      def zero_pool(z, carry):
        pltpu.sync_copy(zbuf, pool_sh.at[pl.ds(pl.multiple_of(z * 8, 8), 8)])
        return carry
      lax.fori_loop(0, PB // 8, zero_pool, 0)

    ninf = jnp.full((L,), -jnp.inf, jnp.float32)
    def pinit(r, carry):
      for f in range(FV):
        pmaxv[r, pl.ds(f * L, L)] = ninf
      return carry
    lax.fori_loop(0, PB, pinit, 0)

    pltpu.sync_copy(bias_hbm.at[cid], biasv)
    plsc.subcore_barrier()

    def do_scale(rowsref, wref):
      def scale(m, c2):
        nv = wref[m]
        for j in range(L):
          sj = lax.index_in_dim(nv, j, 0, keepdims=False)
          e = m * L + j
          for f in range(FVe):
            sl = pl.ds(f * L, L)
            rowsref[e, sl] = rowsref[e, sl] * sj
        return c2
      lax.fori_loop(0, ECHUNK // L, scale, 0)

    off0 = pl.multiple_of(sid * EBASE, ECHUNK)
    offw0 = pl.multiple_of(sid * (EBASE // L), 8)
    pltpu.sync_copy(src2_hbm.at[cid, pl.ds(off0, ECHUNK)], srcv0)
    pltpu.sync_copy(dst_hbm.at[pl.ds(off0, ECHUNK)], dstv0)
    pltpu.sync_copy(w_hbm.at[pl.ds(offw0, ECHUNK // L)], wv0)
    pltpu.async_copy(xs_hbm.at[srcv0], rowsv0, gsem0)

    bufs = ((srcv0, dstv0, wv0, rowsv0, gsem0, ssem0),
            (srcv1, dstv1, wv1, rowsv1, gsem1, ssem1))
    isems = (isem0, isem1)

    def pipe(g2, carry):
      for sslot in (0, 1):
        g = g2 * 2 + sslot
        srcv, dstv, wvs, rowsv, gsem, ssem = bufs[sslot]
        srcn, dstn, wvn, rowsn, gsemn, ssemn = bufs[1 - sslot]
        isn = isems[1 - sslot]
        offn = pl.multiple_of(sid * EBASE + (g + 1) * ECHUNK, ECHUNK)
        offwn = pl.multiple_of(sid * (EBASE // L) + (g + 1) * (ECHUNK // L), 8)

        @pl.when(g + 1 < NCHUNK)
        def _():
          pltpu.async_copy(src2_hbm.at[cid, pl.ds(offn, ECHUNK)], srcn, isn)
          pltpu.async_copy(w_hbm.at[pl.ds(offwn, ECHUNK // L)], wvn, isn)

        @pl.when(g > 0)
        def _():
          pltpu.make_async_copy(rowsn, acc_sh.at[dstn], ssemn).wait()

        @pl.when(g + 1 < NCHUNK)
        def _():
          b = pltpu.async_copy(dst_hbm.at[pl.ds(offn, ECHUNK)], dstn, isn)
          pltpu.make_async_copy(
              src2_hbm.at[cid, pl.ds(offn, ECHUNK)], srcn, isn).wait()
          pltpu.async_copy(xs_hbm.at[srcn], rowsn, gsemn)

        pltpu.make_async_copy(xs_hbm.at[srcv], rowsv, gsem).wait()
        do_scale(rowsv, wvs)

        @pl.when(g + 1 < NCHUNK)
        def _():
          pltpu.make_async_copy(dst_hbm.at[pl.ds(offn, ECHUNK)], dstn, isn).wait()
          pltpu.make_async_copy(w_hbm.at[pl.ds(offwn, ECHUNK // L)], wvn, isn).wait()

        pltpu.async_copy(rowsv, acc_sh.at[dstv], ssem, add=True)
      return carry
    lax.fori_loop(0, NCHUNK // 2, pipe, 0)
    pltpu.make_async_copy(rowsv1, acc_sh.at[dstv1], ssem1).wait()
    plsc.subcore_barrier()

    r0 = pl.multiple_of(sid * RT, 8)
    pltpu.sync_copy(batch_sh.at[pl.ds(r0, RT)], bsm)
    pltpu.sync_copy(dis_sh.at[pl.ds(r0, RT)], dsm)

    def echunk(z, carry):
      rz = pl.multiple_of(r0 + z * 8, 8)
      pltpu.sync_copy(acc_sh.at[pl.ds(rz, 8)], ebuf)
      pltpu.sync_copy(batch_sh.at[pl.ds(rz, 8)], bidx)
      for rr in range(8):
        r = z * 8 + rr
        d = dsm[r]
        for f in range(FVe):
          sl = pl.ds(f * L, L)
          hv = jnp.maximum(ebuf[rr, sl] * d + biasv[sl], 0.0)
          ebuf[rr, sl] = hv
        b = bsm[r]
        for f in range(FVe):
          sl = pl.ds(f * L, L)
          pmaxv[b, sl] = jnp.maximum(pmaxv[b, sl], ebuf[rr, sl])
      pltpu.sync_copy(ebuf, h_hbm.at[cid, pl.ds(rz, 8)])
      pltpu.sync_copy(ebuf, pool_sh.at[bidx], add=True)
      return carry
    lax.fori_loop(0, RT // 8, echunk, 0)
    plsc.subcore_barrier()

    ps0 = pl.multiple_of(sid * B, 8)
    pltpu.sync_copy(pmaxv.at[pl.ds(0, B)], acc_sh.at[pl.ds(ps0, B)])
    plsc.subcore_barrier()

    @pl.when(sid < 8)
    def _():
      s0 = pl.multiple_of(sid * (B // 8), 8)
      for q in range(8):
        for fr in range(FV):
          pmaxv[q, pl.ds(fr * L, L)] = ninf

      def red(pp, carry):
        pltpu.sync_copy(acc_sh.at[pl.ds(pl.multiple_of(pp * B + s0, 8), 8)], ebuf)
        for q in range(8):
          for fr in range(FVe):
            sl = pl.ds(fr * L, L)
            pmaxv[q, sl] = jnp.maximum(pmaxv[q, sl], ebuf[q, sl])
        return carry
      lax.fori_loop(0, NS, red, 0)

      pltpu.sync_copy(pmaxv.at[pl.ds(0, 8)], pmax_hbm.at[cid, pl.ds(s0, 8)])
      pltpu.sync_copy(pool_sh.at[pl.ds(s0, 8)], psum_hbm.at[cid, pl.ds(s0, 8)])

  return pl.kernel(
      body,
      out_type=(
          jax.ShapeDtypeStruct((NC, NP, Hc), jnp.float32),
          jax.ShapeDtypeStruct((NC, B, Hc), jnp.float32),
          jax.ShapeDtypeStruct((NC, B, Hc), jnp.float32),
      ),
      mesh=_MESH,
      scratch_types=[
          pltpu.VMEM_SHARED((ACC, Hc), jnp.float32),
          pltpu.VMEM_SHARED((PB, Hc), jnp.float32),
          pltpu.VMEM_SHARED((NP,), jnp.int32),
          pltpu.VMEM_SHARED((NP,), jnp.float32),
          pltpu.VMEM((8, Hc), jnp.float32),
          pltpu.VMEM((ECHUNK,), jnp.int32),
          pltpu.VMEM((ECHUNK,), jnp.int32),
          pltpu.VMEM((ECHUNK,), jnp.int32),
          pltpu.VMEM((ECHUNK,), jnp.int32),
          pltpu.VMEM((ECHUNK // L, L), jnp.float32),
          pltpu.VMEM((ECHUNK // L, L), jnp.float32),
          pltpu.VMEM((ECHUNK, Hc), jnp.float32),
          pltpu.VMEM((ECHUNK, Hc), jnp.float32),
          pltpu.VMEM((8, Hc), jnp.float32),
          pltpu.VMEM((8,), jnp.int32),
          pltpu.VMEM((Hc,), jnp.float32),
          pltpu.VMEM((PB, Hc), jnp.float32),
          pltpu.SMEM((RT,), jnp.int32),
          pltpu.SMEM((RT,), jnp.float32),
          pltpu.SemaphoreType.DMA,
          pltpu.SemaphoreType.DMA,
          pltpu.SemaphoreType.DMA,
          pltpu.SemaphoreType.DMA,
          pltpu.SemaphoreType.DMA,
          pltpu.SemaphoreType.DMA,
      ],
  )


_prop128 = _make_prop(H // NC, 8)
_prop64w = _make_prop(H // NC, 4)


def _stats_body(x_ref, o_ref):
  xv = x_ref[...]
  s = jnp.sum(xv, axis=0)
  q = jnp.sum(xv * xv, axis=0)
  o_ref[...] = jnp.concatenate(
      [s[None], q[None], jnp.zeros((6, H), jnp.float32)], axis=0)


def _stats(x):
  return pl.pallas_call(
      _stats_body,
      out_shape=jax.ShapeDtypeStruct((8, H), jnp.float32),
  )(x)


def _dis_body(d_ref, o_ref):
  deg = d_ref[0] + d_ref[1]
  o_ref[...] = lax.rsqrt(jnp.maximum(deg, 1e-12))


def _dis(deg2):
  return pl.pallas_call(
      _dis_body,
      out_shape=jax.ShapeDtypeStruct((ACC // 128, 128), jnp.float32),
  )(deg2)


def _mm_body(n, x_ref, w_ref, st_ref, g_ref, bt_ref, dis_ref, o_ref):
  mu = st_ref[0:1] * (1.0 / n)
  msq = st_ref[1:2] * (1.0 / n)
  var = msq - mu * mu
  a = g_ref[...] * lax.rsqrt(var + 1e-5)
  c = bt_ref[...] - mu * a
  t = (x_ref[...] * a + c) * dis_ref[...]
  r = jnp.dot(t, w_ref[...], preferred_element_type=jnp.float32)
  hc = r.shape[1] // 2
  o_ref[0] = r[:, :hc]
  o_ref[1] = r[:, hc:]


def _mm(x, w, stats, gamma, beta, dis):
  n, k = x.shape
  ho = w.shape[1]
  hc = ho // 2
  rblk = 1000
  grid = n // rblk
  return pl.pallas_call(
      functools.partial(_mm_body, float(n)),
      grid=(grid,),
      in_specs=[
          pl.BlockSpec((rblk, k), lambda i: (i, 0)),
          pl.BlockSpec((k, ho), lambda i: (0, 0)),
          pl.BlockSpec((8, k), lambda i: (0, 0)),
          pl.BlockSpec((1, k), lambda i: (0, 0)),
          pl.BlockSpec((1, k), lambda i: (0, 0)),
          pl.BlockSpec((rblk, 1), lambda i: (i, 0)),
      ],
      out_specs=pl.BlockSpec((2, rblk, hc), lambda i: (0, i, 0)),
      out_shape=jax.ShapeDtypeStruct((2, n, hc), jnp.float32),
  )(x, w, stats, gamma, beta, dis)


def _final_body(s1, m1, s2, m2, s3, m3, bt, w1, b1, w2, b2, o_ref):
  seg = lax.broadcasted_iota(jnp.int32, (B, bt.shape[1]), 0)
  eq = (bt[...] == seg).astype(jnp.float32)
  cnt = jnp.sum(eq, axis=1, keepdims=True)
  rc = 1.0 / jnp.maximum(cnt, 1.0)
  xs = jnp.concatenate(
      [s1[...] * rc, m1[...], s2[...] * rc, m2[...], s3[...] * rc, m3[...]],
      axis=1)
  y = jnp.dot(xs, w1[...], preferred_element_type=jnp.float32) + b1[...]
  o_ref[...] = jnp.dot(y, w2[...], preferred_element_type=jnp.float32) + b2[...]


def _final(s1, m1, s2, m2, s3, m3, bt, w1, b1, w2p, b2p):
  return pl.pallas_call(
      _final_body,
      out_shape=jax.ShapeDtypeStruct((B, 128), jnp.float32),
  )(s1, m1, s2, m2, s3, m3, bt, w1, b1, w2p, b2p)


def kernel(x, edge_index, batchsize, edge_weight, gamma, beta,
           W4, b4, W5, b5, W6, b6, W1, b1, W2, b2):
  i32 = edge_index.dtype
  loop = jnp.arange(N, dtype=i32)
  src = jnp.concatenate(
      [edge_index[0], loop, jnp.zeros((ETP - ET,), i32)])
  src2 = jnp.stack([src, src + N])
  dst = jnp.concatenate(
      [edge_index[1], loop, jnp.full((ETP - ET,), DUMP, i32)])
  w = jnp.concatenate(
      [edge_weight, jnp.ones((N,), jnp.float32),
       jnp.zeros((ETP - ET,), jnp.float32)])
  w2d = w.reshape(ETP // L, L)
  batch_p = jnp.concatenate(
      [batchsize.astype(jnp.int32), jnp.full((NP - N,), B, jnp.int32)])
  bt = jnp.concatenate(
      [batchsize.astype(jnp.int32), jnp.full((10240 - N,), B, jnp.int32)]
  ).reshape(1, 10240)

  ones_r = jnp.ones((1, H), jnp.float32)
  zeros_r = jnp.zeros((1, H), jnp.float32)
  stats_id = jnp.concatenate(
      [jnp.zeros((1, H), jnp.float32),
       jnp.full((1, H), float(N) * (1.0 - 1e-5), jnp.float32),
       jnp.zeros((6, H), jnp.float32)], axis=0)

  stats = _stats(x)
  deg2 = _deg(dst, w)
  dis = _dis(deg2.reshape(NC, ACC // 128, 128)).reshape(ACC)
  dis_col = dis[:N].reshape(N, 1)
  dis_np = dis[:NP]

  xw4 = _mm(x, W4, stats, gamma.reshape(1, H), beta.reshape(1, H), dis_col)
  h1, s1, m1 = _prop128(xw4.reshape(2 * N, H // 2), src2, dst, w2d,
                        b4.reshape(2, H // 2), batch_p, dis_np)
  h1c = jnp.concatenate([h1[0, :N], h1[1, :N]], axis=1)

  xw5 = _mm(h1c, W5, stats_id, ones_r, zeros_r, dis_col)
  h2, s2, m2 = _prop128(xw5.reshape(2 * N, H // 2), src2, dst, w2d,
                        b5.reshape(2, H // 2), batch_p, dis_np)
  h2c = jnp.concatenate([h2[0, :N], h2[1, :N]], axis=1)

  w6p = jnp.zeros((H, 2 * H // 2), jnp.float32)
  w6p = w6p.at[:, 0:F // 2].set(W6[:, :F // 2])
  w6p = w6p.at[:, H // 2:H // 2 + F // 2].set(W6[:, F // 2:])
  b6p = jnp.zeros((2, H // 2), jnp.float32)
  b6p = b6p.at[:, :F // 2].set(b6.reshape(2, F // 2))
  xw6 = _mm(h2c, w6p, stats_id, ones_r, zeros_r, dis_col)
  h3, s3, m3 = _prop64w(xw6.reshape(2 * N, H // 2), src2, dst, w2d,
                        b6p, batch_p, dis_np)
  h = jnp.concatenate([h3[0, :N, :F // 2], h3[1, :N, :F // 2]], axis=1)

  s1c = jnp.concatenate([s1[0], s1[1]], axis=1)
  m1c = jnp.concatenate([m1[0], m1[1]], axis=1)
  s2c = jnp.concatenate([s2[0], s2[1]], axis=1)
  m2c = jnp.concatenate([m2[0], m2[1]], axis=1)
  s3c = jnp.concatenate([s3[0][:, :F // 2], s3[1][:, :F // 2]], axis=1)
  m3c = jnp.concatenate([m3[0][:, :F // 2], m3[1][:, :F // 2]], axis=1)

  w2p = jnp.pad(W2, ((0, 0), (0, 127)))
  b2p = jnp.pad(b2.reshape(1, 1), ((0, 0), (0, 127)))
  yfull = _final(s1c, m1c, s2c, m2c, s3c, m3c, bt,
                 W1, b1.reshape(1, H), w2p, b2p)
  y_hat = yfull[:, :1]
  return (h, y_hat)


# speedup vs baseline: 7.5459x; 1.0026x over previous
import functools

import jax
import jax.numpy as jnp
from jax import lax
from jax.experimental import pallas as pl
from jax.experimental.pallas import tpu as pltpu
from jax.experimental.pallas import tpu_sc as plsc

N = 10000
E = 160000
H = 256
F = 128
B = 64

NC = 2
NS = 16
L = 16

ET = E + N
ETP = 172032
ECHUNK = 128
NCHUNK = ETP // NS // ECHUNK
EBASE = ETP // NS
WCHUNK = ETP // (NC * NS) // ECHUNK

NP = 10112
ACC = 10240
DUMP = NP
RT = NP // NS
ZR = ACC // NS
PB = 72

_MESH = plsc.VectorSubcoreMesh(
    core_axis_name="c", subcore_axis_name="s", num_cores=NC, num_subcores=NS)


def _deg_body(dst_hbm, w_hbm, deg_hbm, deg_sh, zbuf, idxv, wv, gsem):
  cid = lax.axis_index("c")
  sid = lax.axis_index("s")
  wid = sid * NC + cid

  zv = jnp.zeros((L,), jnp.float32)
  for i in range(ZR // L):
    zbuf[pl.ds(i * L, L)] = zv
  pltpu.sync_copy(zbuf, deg_sh.at[pl.ds(pl.multiple_of(sid * ZR, 8), ZR)])
  plsc.subcore_barrier()

  def deg_chunk(g, carry):
    off = pl.multiple_of(wid * (ETP // (NC * NS)) + g * ECHUNK, ECHUNK)
    pltpu.sync_copy(dst_hbm.at[pl.ds(off, ECHUNK)], idxv)
    pltpu.sync_copy(w_hbm.at[pl.ds(off, ECHUNK)], wv)
    pltpu.sync_copy(wv, deg_sh.at[idxv], add=True)
    return carry
  lax.fori_loop(0, WCHUNK, deg_chunk, 0)
  plsc.subcore_barrier()

  zr0 = pl.multiple_of(sid * ZR, 8)
  pltpu.sync_copy(deg_sh.at[pl.ds(zr0, ZR)],
                  deg_hbm.at[cid, pl.ds(zr0, ZR)])


_deg = pl.kernel(
    _deg_body,
    out_type=jax.ShapeDtypeStruct((NC, ACC), jnp.float32),
    mesh=_MESH,
    scratch_types=[
        pltpu.VMEM_SHARED((ACC,), jnp.float32),
        pltpu.VMEM((ZR,), jnp.float32),
        pltpu.VMEM((ECHUNK,), jnp.int32),
        pltpu.VMEM((ECHUNK,), jnp.float32),
        pltpu.SemaphoreType.DMA,
    ],
)


def _make_prop(Hc, FVe):

  def body(xs_hbm, src2_hbm, dst_hbm, w_hbm, bias_hbm, batch_hbm, dis_hbm,
           h_hbm, psum_hbm, pmax_hbm,
           acc_sh, pool_sh, batch_sh, dis_sh,
           zbuf, srcv0, srcv1, dstv0, dstv1, wv0, wv1, rowsv0, rowsv1,
           ebuf, bidx, biasv, pmaxv,
           bsm, dsm, isem0, isem1, gsem0, gsem1, ssem0, ssem1):
    cid = lax.axis_index("c")
    sid = lax.axis_index("s")
    FV = Hc // L

    @pl.when(sid == 0)
    def _():
      pltpu.sync_copy(batch_hbm, batch_sh)
      pltpu.sync_copy(dis_hbm, dis_sh)

    zv = jnp.zeros((L,), jnp.float32)
    for r in range(8):
      for f in range(FV):
        zbuf[r, pl.ds(f * L, L)] = zv

    def zero_chunk(z, carry):
      pltpu.sync_copy(zbuf, acc_sh.at[pl.ds(pl.multiple_of(sid * ZR + z * 8, 8), 8)])
      return carry
    lax.fori_loop(0, ZR // 8, zero_chunk, 0)

    @pl.when(sid == 0)
    def _():
      def zero_pool(z, carry):
        pltpu.sync_copy(zbuf, pool_sh.at[pl.ds(pl.multiple_of(z * 8, 8), 8)])
        return carry
      lax.fori_loop(0, PB // 8, zero_pool, 0)

    ninf = jnp.full((L,), -jnp.inf, jnp.float32)
    def pinit(r, carry):
      for f in range(FV):
        pmaxv[r, pl.ds(f * L, L)] = ninf
      return carry
    lax.fori_loop(0, PB, pinit, 0)

    pltpu.sync_copy(bias_hbm.at[cid], biasv)
    plsc.subcore_barrier()

    def do_scale(rowsref, wref):
      def scale(mi, c2):
        for u in range(2):
          m = mi * 2 + u
          nv = wref[m]
          for j in range(L):
            sj = lax.index_in_dim(nv, j, 0, keepdims=False)
            e = m * L + j
            for f in range(FVe):
              sl = pl.ds(f * L, L)
              rowsref[e, sl] = rowsref[e, sl] * sj
        return c2
      lax.fori_loop(0, ECHUNK // L // 2, scale, 0)

    off0 = pl.multiple_of(sid * EBASE, ECHUNK)
    offw0 = pl.multiple_of(sid * (EBASE // L), 8)
    pltpu.sync_copy(src2_hbm.at[cid, pl.ds(off0, ECHUNK)], srcv0)
    pltpu.sync_copy(dst_hbm.at[pl.ds(off0, ECHUNK)], dstv0)
    pltpu.sync_copy(w_hbm.at[pl.ds(offw0, ECHUNK // L)], wv0)
    pltpu.async_copy(xs_hbm.at[srcv0], rowsv0, gsem0)

    bufs = ((srcv0, dstv0, wv0, rowsv0, gsem0, ssem0),
            (srcv1, dstv1, wv1, rowsv1, gsem1, ssem1))
    isems = (isem0, isem1)

    def pipe(g2, carry):
      for sslot in (0, 1):
        g = g2 * 2 + sslot
        srcv, dstv, wvs, rowsv, gsem, ssem = bufs[sslot]
        srcn, dstn, wvn, rowsn, gsemn, ssemn = bufs[1 - sslot]
        isn = isems[1 - sslot]
        offn = pl.multiple_of(sid * EBASE + (g + 1) * ECHUNK, ECHUNK)
        offwn = pl.multiple_of(sid * (EBASE // L) + (g + 1) * (ECHUNK // L), 8)

        @pl.when(g + 1 < NCHUNK)
        def _():
          pltpu.async_copy(src2_hbm.at[cid, pl.ds(offn, ECHUNK)], srcn, isn)
          pltpu.async_copy(w_hbm.at[pl.ds(offwn, ECHUNK // L)], wvn, isn)

        @pl.when(g > 0)
        def _():
          pltpu.make_async_copy(rowsn, acc_sh.at[dstn], ssemn).wait()

        @pl.when(g + 1 < NCHUNK)
        def _():
          b = pltpu.async_copy(dst_hbm.at[pl.ds(offn, ECHUNK)], dstn, isn)
          pltpu.make_async_copy(
              src2_hbm.at[cid, pl.ds(offn, ECHUNK)], srcn, isn).wait()
          pltpu.async_copy(xs_hbm.at[srcn], rowsn, gsemn)

        pltpu.make_async_copy(xs_hbm.at[srcv], rowsv, gsem).wait()
        do_scale(rowsv, wvs)

        @pl.when(g + 1 < NCHUNK)
        def _():
          pltpu.make_async_copy(dst_hbm.at[pl.ds(offn, ECHUNK)], dstn, isn).wait()
          pltpu.make_async_copy(w_hbm.at[pl.ds(offwn, ECHUNK // L)], wvn, isn).wait()

        pltpu.async_copy(rowsv, acc_sh.at[dstv], ssem, add=True)
      return carry
    lax.fori_loop(0, NCHUNK // 2, pipe, 0)
    pltpu.make_async_copy(rowsv1, acc_sh.at[dstv1], ssem1).wait()
    plsc.subcore_barrier()

    r0 = pl.multiple_of(sid * RT, 8)
    pltpu.sync_copy(batch_sh.at[pl.ds(r0, RT)], bsm)
    pltpu.sync_copy(dis_sh.at[pl.ds(r0, RT)], dsm)

    def echunk(z, carry):
      rz = pl.multiple_of(r0 + z * 8, 8)
      pltpu.sync_copy(acc_sh.at[pl.ds(rz, 8)], ebuf)
      pltpu.sync_copy(batch_sh.at[pl.ds(rz, 8)], bidx)
      for rr in range(8):
        r = z * 8 + rr
        d = dsm[r]
        for f in range(FVe):
          sl = pl.ds(f * L, L)
          hv = jnp.maximum(ebuf[rr, sl] * d + biasv[sl], 0.0)
          ebuf[rr, sl] = hv
        b = bsm[r]
        for f in range(FVe):
          sl = pl.ds(f * L, L)
          pmaxv[b, sl] = jnp.maximum(pmaxv[b, sl], ebuf[rr, sl])
      pltpu.sync_copy(ebuf, h_hbm.at[cid, pl.ds(rz, 8)])
      pltpu.sync_copy(ebuf, pool_sh.at[bidx], add=True)
      return carry
    lax.fori_loop(0, RT // 8, echunk, 0)
    plsc.subcore_barrier()

    ps0 = pl.multiple_of(sid * B, 8)
    pltpu.sync_copy(pmaxv.at[pl.ds(0, B)], acc_sh.at[pl.ds(ps0, B)])
    plsc.subcore_barrier()

    @pl.when(sid < 8)
    def _():
      s0 = pl.multiple_of(sid * (B // 8), 8)
      for q in range(8):
        for fr in range(FV):
          pmaxv[q, pl.ds(fr * L, L)] = ninf

      def red(pp, carry):
        pltpu.sync_copy(acc_sh.at[pl.ds(pl.multiple_of(pp * B + s0, 8), 8)], ebuf)
        for q in range(8):
          for fr in range(FVe):
            sl = pl.ds(fr * L, L)
            pmaxv[q, sl] = jnp.maximum(pmaxv[q, sl], ebuf[q, sl])
        return carry
      lax.fori_loop(0, NS, red, 0)

      pltpu.sync_copy(pmaxv.at[pl.ds(0, 8)], pmax_hbm.at[cid, pl.ds(s0, 8)])
      pltpu.sync_copy(pool_sh.at[pl.ds(s0, 8)], psum_hbm.at[cid, pl.ds(s0, 8)])

  return pl.kernel(
      body,
      out_type=(
          jax.ShapeDtypeStruct((NC, NP, Hc), jnp.float32),
          jax.ShapeDtypeStruct((NC, B, Hc), jnp.float32),
          jax.ShapeDtypeStruct((NC, B, Hc), jnp.float32),
      ),
      mesh=_MESH,
      scratch_types=[
          pltpu.VMEM_SHARED((ACC, Hc), jnp.float32),
          pltpu.VMEM_SHARED((PB, Hc), jnp.float32),
          pltpu.VMEM_SHARED((NP,), jnp.int32),
          pltpu.VMEM_SHARED((NP,), jnp.float32),
          pltpu.VMEM((8, Hc), jnp.float32),
          pltpu.VMEM((ECHUNK,), jnp.int32),
          pltpu.VMEM((ECHUNK,), jnp.int32),
          pltpu.VMEM((ECHUNK,), jnp.int32),
          pltpu.VMEM((ECHUNK,), jnp.int32),
          pltpu.VMEM((ECHUNK // L, L), jnp.float32),
          pltpu.VMEM((ECHUNK // L, L), jnp.float32),
          pltpu.VMEM((ECHUNK, Hc), jnp.float32),
          pltpu.VMEM((ECHUNK, Hc), jnp.float32),
          pltpu.VMEM((8, Hc), jnp.float32),
          pltpu.VMEM((8,), jnp.int32),
          pltpu.VMEM((Hc,), jnp.float32),
          pltpu.VMEM((PB, Hc), jnp.float32),
          pltpu.SMEM((RT,), jnp.int32),
          pltpu.SMEM((RT,), jnp.float32),
          pltpu.SemaphoreType.DMA,
          pltpu.SemaphoreType.DMA,
          pltpu.SemaphoreType.DMA,
          pltpu.SemaphoreType.DMA,
          pltpu.SemaphoreType.DMA,
          pltpu.SemaphoreType.DMA,
      ],
  )


_prop128 = _make_prop(H // NC, 8)
_prop64w = _make_prop(H // NC, 4)


def _stats_body(x_ref, o_ref):
  xv = x_ref[...]
  s = jnp.sum(xv, axis=0)
  q = jnp.sum(xv * xv, axis=0)
  o_ref[...] = jnp.concatenate(
      [s[None], q[None], jnp.zeros((6, H), jnp.float32)], axis=0)


def _stats(x):
  return pl.pallas_call(
      _stats_body,
      out_shape=jax.ShapeDtypeStruct((8, H), jnp.float32),
  )(x)


def _dis_body(d_ref, o_ref):
  deg = d_ref[0] + d_ref[1]
  o_ref[...] = lax.rsqrt(jnp.maximum(deg, 1e-12))


def _dis(deg2):
  return pl.pallas_call(
      _dis_body,
      out_shape=jax.ShapeDtypeStruct((ACC // 128, 128), jnp.float32),
  )(deg2)


def _mm_body(n, x_ref, w_ref, st_ref, g_ref, bt_ref, dis_ref, o_ref):
  mu = st_ref[0:1] * (1.0 / n)
  msq = st_ref[1:2] * (1.0 / n)
  var = msq - mu * mu
  a = g_ref[...] * lax.rsqrt(var + 1e-5)
  c = bt_ref[...] - mu * a
  t = (x_ref[...] * a + c) * dis_ref[...]
  r = jnp.dot(t, w_ref[...], preferred_element_type=jnp.float32)
  hc = r.shape[1] // 2
  o_ref[0] = r[:, :hc]
  o_ref[1] = r[:, hc:]


def _mm(x, w, stats, gamma, beta, dis):
  n, k = x.shape
  ho = w.shape[1]
  hc = ho // 2
  rblk = 1000
  grid = n // rblk
  return pl.pallas_call(
      functools.partial(_mm_body, float(n)),
      grid=(grid,),
      in_specs=[
          pl.BlockSpec((rblk, k), lambda i: (i, 0)),
          pl.BlockSpec((k, ho), lambda i: (0, 0)),
          pl.BlockSpec((8, k), lambda i: (0, 0)),
          pl.BlockSpec((1, k), lambda i: (0, 0)),
          pl.BlockSpec((1, k), lambda i: (0, 0)),
          pl.BlockSpec((rblk, 1), lambda i: (i, 0)),
      ],
      out_specs=pl.BlockSpec((2, rblk, hc), lambda i: (0, i, 0)),
      out_shape=jax.ShapeDtypeStruct((2, n, hc), jnp.float32),
  )(x, w, stats, gamma, beta, dis)


def _final_body(s1, m1, s2, m2, s3, m3, bt, w1, b1, w2, b2, o_ref):
  seg = lax.broadcasted_iota(jnp.int32, (B, bt.shape[1]), 0)
  eq = (bt[...] == seg).astype(jnp.float32)
  cnt = jnp.sum(eq, axis=1, keepdims=True)
  rc = 1.0 / jnp.maximum(cnt, 1.0)
  xs = jnp.concatenate(
      [s1[...] * rc, m1[...], s2[...] * rc, m2[...], s3[...] * rc, m3[...]],
      axis=1)
  y = jnp.dot(xs, w1[...], preferred_element_type=jnp.float32) + b1[...]
  o_ref[...] = jnp.dot(y, w2[...], preferred_element_type=jnp.float32) + b2[...]


def _final(s1, m1, s2, m2, s3, m3, bt, w1, b1, w2p, b2p):
  return pl.pallas_call(
      _final_body,
      out_shape=jax.ShapeDtypeStruct((B, 128), jnp.float32),
  )(s1, m1, s2, m2, s3, m3, bt, w1, b1, w2p, b2p)


def kernel(x, edge_index, batchsize, edge_weight, gamma, beta,
           W4, b4, W5, b5, W6, b6, W1, b1, W2, b2):
  i32 = edge_index.dtype
  loop = jnp.arange(N, dtype=i32)
  src = jnp.concatenate(
      [edge_index[0], loop, jnp.zeros((ETP - ET,), i32)])
  src2 = jnp.stack([src, src + N])
  dst = jnp.concatenate(
      [edge_index[1], loop, jnp.full((ETP - ET,), DUMP, i32)])
  w = jnp.concatenate(
      [edge_weight, jnp.ones((N,), jnp.float32),
       jnp.zeros((ETP - ET,), jnp.float32)])
  w2d = w.reshape(ETP // L, L)
  batch_p = jnp.concatenate(
      [batchsize.astype(jnp.int32), jnp.full((NP - N,), B, jnp.int32)])
  bt = jnp.concatenate(
      [batchsize.astype(jnp.int32), jnp.full((10240 - N,), B, jnp.int32)]
  ).reshape(1, 10240)

  ones_r = jnp.ones((1, H), jnp.float32)
  zeros_r = jnp.zeros((1, H), jnp.float32)
  stats_id = jnp.concatenate(
      [jnp.zeros((1, H), jnp.float32),
       jnp.full((1, H), float(N) * (1.0 - 1e-5), jnp.float32),
       jnp.zeros((6, H), jnp.float32)], axis=0)

  stats = _stats(x)
  deg2 = _deg(dst, w)
  dis = _dis(deg2.reshape(NC, ACC // 128, 128)).reshape(ACC)
  dis_col = dis[:N].reshape(N, 1)
  dis_np = dis[:NP]

  xw4 = _mm(x, W4, stats, gamma.reshape(1, H), beta.reshape(1, H), dis_col)
  h1, s1, m1 = _prop128(xw4.reshape(2 * N, H // 2), src2, dst, w2d,
                        b4.reshape(2, H // 2), batch_p, dis_np)
  h1c = jnp.concatenate([h1[0, :N], h1[1, :N]], axis=1)

  xw5 = _mm(h1c, W5, stats_id, ones_r, zeros_r, dis_col)
  h2, s2, m2 = _prop128(xw5.reshape(2 * N, H // 2), src2, dst, w2d,
                        b5.reshape(2, H // 2), batch_p, dis_np)
  h2c = jnp.concatenate([h2[0, :N], h2[1, :N]], axis=1)

  w6p = jnp.zeros((H, 2 * H // 2), jnp.float32)
  w6p = w6p.at[:, 0:F // 2].set(W6[:, :F // 2])
  w6p = w6p.at[:, H // 2:H // 2 + F // 2].set(W6[:, F // 2:])
  b6p = jnp.zeros((2, H // 2), jnp.float32)
  b6p = b6p.at[:, :F // 2].set(b6.reshape(2, F // 2))
  xw6 = _mm(h2c, w6p, stats_id, ones_r, zeros_r, dis_col)
  h3, s3, m3 = _prop64w(xw6.reshape(2 * N, H // 2), src2, dst, w2d,
                        b6p, batch_p, dis_np)
  h = jnp.concatenate([h3[0, :N, :F // 2], h3[1, :N, :F // 2]], axis=1)

  s1c = jnp.concatenate([s1[0], s1[1]], axis=1)
  m1c = jnp.concatenate([m1[0], m1[1]], axis=1)
  s2c = jnp.concatenate([s2[0], s2[1]], axis=1)
  m2c = jnp.concatenate([m2[0], m2[1]], axis=1)
  s3c = jnp.concatenate([s3[0][:, :F // 2], s3[1][:, :F // 2]], axis=1)
  m3c = jnp.concatenate([m3[0][:, :F // 2], m3[1][:, :F // 2]], axis=1)

  w2p = jnp.pad(W2, ((0, 0), (0, 127)))
  b2p = jnp.pad(b2.reshape(1, 1), ((0, 0), (0, 127)))
  yfull = _final(s1c, m1c, s2c, m2c, s3c, m3c, bt,
                 W1, b1.reshape(1, H), w2p, b2p)
  y_hat = yfull[:, :1]
  return (h, y_hat)
